# Initial kernel scaffold; baseline (speedup 1.0000x reference)
import jax
import jax.numpy as jnp
from jax.experimental import pallas as pl


def kernel(reads, table):
    raise NotImplementedError("write your pallas kernel here")


# speedup vs baseline: 215.6062x; 215.6062x over previous
import functools

import jax
import jax.numpy as jnp
from jax.experimental import pallas as pl
from jax.experimental.pallas import tpu as pltpu


def _enc_block(reads_ref, table_ref, out_ref, *, inv_len):
    r = reads_ref[...]
    b0 = (r & 1).astype(jnp.float32)
    b1 = (r >> 1).astype(jnp.float32)
    s0 = jnp.sum(b0, axis=1, keepdims=True)
    s1 = jnp.sum(b1, axis=1, keepdims=True)
    c3 = jnp.sum(b0 * b1, axis=1, keepdims=True)
    c1 = s0 - c3
    c2 = s1 - c3
    c0 = jnp.float32(r.shape[1]) - c1 - c2 - c3
    t = table_ref[...]
    acc = (c0 * t[0, :][None, :] + c1 * t[1, :][None, :]
           + c2 * t[2, :][None, :] + c3 * t[3, :][None, :])
    out_ref[...] = acc * inv_len


def kernel(reads, table):
    reads = reads.astype(jnp.int32)
    b, l = reads.shape
    k, d = table.shape
    tb = 512
    grid = (b // tb,)
    return pl.pallas_call(
        functools.partial(_enc_block, inv_len=1.0 / l),
        grid=grid,
        in_specs=[
            pl.BlockSpec((tb, l), lambda i: (i, 0)),
            pl.BlockSpec((k, d), lambda i: (0, 0)),
        ],
        out_specs=pl.BlockSpec((tb, d), lambda i: (i, 0)),
        out_shape=jax.ShapeDtypeStruct((b, d), jnp.float32),
    )(reads, table)
